# baseline (speedup 1.0000x reference)
import jax
import jax.numpy as jnp
from jax import lax
from jax.experimental import pallas as pl
from jax.experimental.pallas import tpu as pltpu
from jax.experimental.pallas import tpu_sc as plsc

D = 128
B, S = 1024, 200
N = B * S
NW = 32
RPW = B // NW
CA, CB = 128, S - 128
NSLOT = 4
GR = 2


def _embed_body(tok_hbm, tab_hbm, out_hbm, idx_v, rows_a, rows_b, *sems):
    ga = sems[0:NSLOT]
    gb = sems[NSLOT:2 * NSLOT]
    sa = sems[2 * NSLOT:3 * NSLOT]
    sb = sems[3 * NSLOT:4 * NSLOT]
    wid = lax.axis_index("s") * 2 + lax.axis_index("c")
    base = wid * RPW * S

    pltpu.sync_copy(tok_hbm.at[pl.ds(wid * RPW, RPW)], idx_v)

    def gather_row(r, s):
        pltpu.async_copy(
            tab_hbm.at[idx_v.at[r, pl.ds(0, CA)]], rows_a.at[s], ga[s]
        )
        pltpu.async_copy(
            tab_hbm.at[idx_v.at[r, pl.ds(CA, CB)]], rows_b.at[s], gb[s]
        )

    def wait_gathers(s):
        pltpu.make_async_copy(
            tab_hbm.at[idx_v.at[0, pl.ds(0, CA)]], rows_a.at[s], ga[s]
        ).wait()
        pltpu.make_async_copy(
            tab_hbm.at[idx_v.at[0, pl.ds(CA, CB)]], rows_b.at[s], gb[s]
        ).wait()

    def wait_stores(s):
        pltpu.make_async_copy(
            rows_a.at[s], out_hbm.at[pl.ds(0, CA)], sa[s]
        ).wait()
        pltpu.make_async_copy(
            rows_b.at[s], out_hbm.at[pl.ds(0, CB)], sb[s]
        ).wait()

    for r in range(GR):
        gather_row(r, r)

    def body(i, carry):
        for ss in range(NSLOT):
            r = i * NSLOT + ss
            wait_gathers(ss)
            pltpu.async_copy(
                rows_a.at[ss], out_hbm.at[pl.ds(base + r * S, CA)], sa[ss]
            )
            pltpu.async_copy(
                rows_b.at[ss], out_hbm.at[pl.ds(base + r * S + CA, CB)], sb[ss]
            )
            nr = r + GR
            ns = (ss + GR) % NSLOT

            @pl.when(nr < RPW)
            def _():
                @pl.when(nr - NSLOT >= 0)
                def _():
                    wait_stores(ns)

                gather_row(nr, ns)
        return carry

    lax.fori_loop(0, RPW // NSLOT, body, 0)

    for ss in range(NSLOT):
        wait_stores(ss)


def kernel(tokens, table):
    mesh = plsc.VectorSubcoreMesh(core_axis_name="c", subcore_axis_name="s")
    out = pl.kernel(
        _embed_body,
        out_type=jax.ShapeDtypeStruct((N, D), jnp.float32),
        mesh=mesh,
        scratch_types=[
            pltpu.VMEM((RPW, S), jnp.int32),
            pltpu.VMEM((NSLOT, CA, D), jnp.float32),
            pltpu.VMEM((NSLOT, CB, D), jnp.float32),
        ] + [pltpu.SemaphoreType.DMA] * (4 * NSLOT),
    )(tokens, table)
    return out.reshape(B, S, D)


# speedup vs baseline: 1.0021x; 1.0021x over previous
import jax
import jax.numpy as jnp
from jax import lax
from jax.experimental import pallas as pl
from jax.experimental.pallas import tpu as pltpu
from jax.experimental.pallas import tpu_sc as plsc

D = 128
B, S = 1024, 200
N = B * S
NW = 32
RPW = B // NW
CA, CB = 128, S - 128
NSLOT = 4
GR = 2


def _embed_body(tok_hbm, tab_hbm, out_hbm, idx_v, rows_a, rows_b, *sems):
    ga = sems[0:NSLOT]
    gb = sems[NSLOT:2 * NSLOT]
    sa = sems[2 * NSLOT:3 * NSLOT]
    sb = sems[3 * NSLOT:4 * NSLOT]
    wid = lax.axis_index("s") * 2 + lax.axis_index("c")
    base = wid * RPW * S

    pltpu.sync_copy(tok_hbm.at[pl.ds(wid * RPW, RPW)], idx_v)

    def gather_row(r, s):
        pltpu.async_copy(
            tab_hbm.at[idx_v.at[r, pl.ds(0, CA)]], rows_a.at[s], ga[s]
        )
        pltpu.async_copy(
            tab_hbm.at[idx_v.at[r, pl.ds(CA, CB)]], rows_b.at[s], gb[s]
        )

    def wait_gathers(s):
        pltpu.make_async_copy(
            tab_hbm.at[idx_v.at[0, pl.ds(0, CA)]], rows_a.at[s], ga[s]
        ).wait()
        pltpu.make_async_copy(
            tab_hbm.at[idx_v.at[0, pl.ds(CA, CB)]], rows_b.at[s], gb[s]
        ).wait()

    def wait_stores(s):
        pltpu.make_async_copy(
            rows_a.at[s], out_hbm.at[pl.ds(0, CA)], sa[s]
        ).wait()
        pltpu.make_async_copy(
            rows_b.at[s], out_hbm.at[pl.ds(0, CB)], sb[s]
        ).wait()

    for r in range(GR):
        gather_row(r, r)

    def body(i, carry):
        for ss in range(NSLOT):
            r = i * NSLOT + ss
            wait_gathers(ss)
            pltpu.async_copy(
                rows_a.at[ss], out_hbm.at[pl.ds(base + r * S, CA)], sa[ss]
            )
            pltpu.async_copy(
                rows_b.at[ss], out_hbm.at[pl.ds(base + r * S + CA, CB)], sb[ss]
            )
            nr = r + GR
            ns = (ss + GR) % NSLOT

            @pl.when(nr < RPW)
            def _():
                @pl.when(nr - NSLOT >= 0)
                def _():
                    wait_stores(ns)

                gather_row(nr, ns)
        return carry

    lax.fori_loop(0, RPW // NSLOT, body, 0)

    for ss in range(NSLOT):
        wait_stores(ss)


def kernel(tokens, table):
    mesh = plsc.VectorSubcoreMesh(core_axis_name="c", subcore_axis_name="s")
    out = pl.kernel(
        _embed_body,
        out_type=jax.ShapeDtypeStruct((N, D), jnp.float32),
        mesh=mesh,
        scratch_types=[
            pltpu.VMEM((RPW, S), jnp.int32),
            pltpu.VMEM((NSLOT, CA, D), jnp.float32),
            pltpu.VMEM((NSLOT, CB, D), jnp.float32),
        ] + [pltpu.SemaphoreType.DMA] * (4 * NSLOT),
        compiler_params=pltpu.CompilerParams(use_tc_tiling_on_sc=True),
    )(tokens, table)
    return out.reshape(B, S, D)
